# baseline (speedup 1.0000x reference)
import functools

import jax
import jax.numpy as jnp
from jax import lax
from jax.experimental import pallas as pl
from jax.experimental.pallas import tpu as pltpu
from jax.experimental.pallas import tpu_sc as plsc

_G = 16


@functools.lru_cache(maxsize=None)
def _make_gather_kernel(B, D, NC, NS):
    NW = NC * NS
    b_per_w = B // NW
    n_rounds = b_per_w // _G
    mesh = plsc.VectorSubcoreMesh(core_axis_name="c", subcore_axis_name="s")

    @functools.partial(
        pl.kernel,
        mesh=mesh,
        out_type=(
            jax.ShapeDtypeStruct((B, D), jnp.float32),
            jax.ShapeDtypeStruct((B, D), jnp.float32),
        ),
        scratch_types=[
            pltpu.VMEM((b_per_w,), jnp.int32),
            pltpu.VMEM((b_per_w,), jnp.int32),
            pltpu.VMEM((_G, 8, D), jnp.float32),
            pltpu.VMEM((_G, 8, D), jnp.float32),
            pltpu.VMEM((_G, D), jnp.float32),
            pltpu.VMEM((_G, D), jnp.float32),
            pltpu.SemaphoreType.DMA,
        ],
    )
    def gather_scale(users_hbm, items_hbm, ut3_hbm, it3_hbm,
                     out_u_hbm, out_i_hbm,
                     uidx_v, iidx_v, tiles_u, tiles_i, orow_u, orow_i, sem):
        wid = lax.axis_index("s") * NC + lax.axis_index("c")
        base = wid * b_per_w
        pltpu.sync_copy(users_hbm.at[pl.ds(base, b_per_w)], uidx_v)
        pltpu.sync_copy(items_hbm.at[pl.ds(base, b_per_w)], iidx_v)

        def round_body(g, carry):
            uv = uidx_v[pl.ds(g * _G, _G)]
            iv = iidx_v[pl.ds(g * _G, _G)]
            ut_tiles = uv >> 3
            it_tiles = iv >> 3
            copies = []
            for s in range(_G):
                copies.append(pltpu.async_copy(
                    ut3_hbm.at[ut_tiles[s]], tiles_u.at[s], sem))
                copies.append(pltpu.async_copy(
                    it3_hbm.at[it_tiles[s]], tiles_i.at[s], sem))
            for c in copies:
                c.wait()
            for s in range(_G):
                usub = uv[s] & 7
                isub = iv[s] & 7
                for k in range(D // 16):
                    sl = pl.ds(k * 16, 16)
                    orow_u[s, sl] = tiles_u[s, usub, sl] * 0.25
                    orow_i[s, sl] = tiles_i[s, isub, sl] * 0.25
            pltpu.sync_copy(orow_u, out_u_hbm.at[pl.ds(base + g * _G, _G)])
            pltpu.sync_copy(orow_i, out_i_hbm.at[pl.ds(base + g * _G, _G)])
            return carry

        lax.fori_loop(0, n_rounds, round_body, 0)

    return gather_scale


def kernel(users, items, user_table, item_table, edge_user, edge_item):
    B = users.shape[0]
    D = user_table.shape[1]
    info = plsc.get_sparse_core_info()
    fn = _make_gather_kernel(B, D, info.num_cores, info.num_subcores)
    ut3 = user_table.reshape(user_table.shape[0] // 8, 8, D)
    it3 = item_table.reshape(item_table.shape[0] // 8, 8, D)
    return fn(users, items, ut3, it3)
